# Initial kernel scaffold; baseline (speedup 1.0000x reference)
import jax
import jax.numpy as jnp
from jax.experimental import pallas as pl


def kernel(x, types, W, b):
    raise NotImplementedError("write your pallas kernel here")


# speedup vs baseline: 2.3615x; 2.3615x over previous
import functools

import jax
import jax.numpy as jnp
from jax import lax
from jax.experimental import pallas as pl
from jax.experimental.pallas import tpu as pltpu
from jax.experimental.pallas import tpu_sc as plsc

NUM_TYPES = 8
D = 1024
B = 8192
BM = 256
NBLK = B // BM
MAX_WORK = NBLK + NUM_TYPES - 1
SUB = 64
LANES = 128
NW = 32
ROWS_PER_W = B // NW
CHUNK = 64


def _routing_body(types_ref, pos_ref, starts_ref):
    t = types_ref[...]
    r128 = lax.broadcasted_iota(jnp.int32, (LANES, LANES), 0)
    c128 = lax.broadcasted_iota(jnp.int32, (LANES, LANES), 1)
    upper_incl = (r128 <= c128).astype(jnp.float32)
    ones_l = jnp.ones((LANES, LANES), dtype=jnp.float32)
    r64 = lax.broadcasted_iota(jnp.int32, (SUB, SUB), 0)
    c64 = lax.broadcasted_iota(jnp.int32, (SUB, SUB), 1)
    strict_lower = (c64 < r64).astype(jnp.float32)

    start = jnp.float32(0.0)
    pos_f = jnp.zeros((SUB, LANES), dtype=jnp.float32)
    start_rows = []
    for tt in range(NUM_TYPES):
        m = (t == tt).astype(jnp.float32)
        rows_before = lax.dot(strict_lower, m, precision=lax.Precision.HIGHEST)
        incl = (
            lax.dot(rows_before, ones_l, precision=lax.Precision.HIGHEST)
            + lax.dot(m, upper_incl, precision=lax.Precision.HIGHEST)
        )
        pos_f = pos_f + m * (start + incl - 1.0)
        start_rows.append(jnp.broadcast_to(jnp.reshape(start, (1, 1)), (1, LANES)))
        start = start + jnp.sum(m)
    pos_ref[...] = pos_f.astype(jnp.int32)
    starts_ref[...] = jnp.concatenate(start_rows, axis=0).astype(jnp.int32)


def _routing(types2d):
    return pl.pallas_call(
        _routing_body,
        out_shape=(
            jax.ShapeDtypeStruct((SUB, LANES), jnp.int32),
            jax.ShapeDtypeStruct((NUM_TYPES, LANES), jnp.int32),
        ),
    )(types2d)


def _gmm_body(rb_ref, tb_ref, gs_ref, ge_ref, x_ref, w_ref, b_ref, out_ref):
    w = pl.program_id(0)
    rb = rb_ref[w]
    prev_rb = rb_ref[jnp.maximum(w - 1, 0)]
    is_first = jnp.logical_or(w == 0, rb != prev_rb)

    @pl.when(is_first)
    def _init():
        out_ref[...] = jnp.zeros_like(out_ref)

    gs = gs_ref[w]
    ge = ge_ref[w]

    @pl.when(gs < ge)
    def _compute():
        xb = x_ref[...].astype(jnp.bfloat16)
        wb = w_ref[0].astype(jnp.bfloat16)
        acc = lax.dot_general(
            xb, wb, (((1,), (1,)), ((), ())),
            preferred_element_type=jnp.float32,
        )
        rows = rb * BM + lax.broadcasted_iota(jnp.int32, (BM, 1), 0)
        mask = jnp.logical_and(rows >= gs, rows < ge)
        out_ref[...] += jnp.where(mask, acc + b_ref[0], 0.0)


def _grouped_matmul(rb, tb, gs, ge, x_sorted, W, b):
    grid_spec = pltpu.PrefetchScalarGridSpec(
        num_scalar_prefetch=4,
        grid=(MAX_WORK,),
        in_specs=[
            pl.BlockSpec((BM, D), lambda w, rb, tb, gs, ge: (rb[w], 0)),
            pl.BlockSpec((1, D, D), lambda w, rb, tb, gs, ge: (tb[w], 0, 0)),
            pl.BlockSpec((1, 1, D), lambda w, rb, tb, gs, ge: (tb[w], 0, 0)),
        ],
        out_specs=pl.BlockSpec((BM, D), lambda w, rb, tb, gs, ge: (rb[w], 0)),
    )
    return pl.pallas_call(
        _gmm_body,
        grid_spec=grid_spec,
        out_shape=jax.ShapeDtypeStruct((B, D), jnp.float32),
        compiler_params=pltpu.CompilerParams(
            dimension_semantics=("arbitrary",),
        ),
    )(rb, tb, gs, ge, x_sorted, W, b.reshape(NUM_TYPES, 1, D))


def _sc_scatter_body(x_hbm, pos_hbm, out_hbm, idx_v, rows_v, sem):
    wid = lax.axis_index("s") * 2 + lax.axis_index("c")
    base = wid * ROWS_PER_W
    for k in range(ROWS_PER_W // CHUNK):
        off = base + k * CHUNK
        pltpu.sync_copy(pos_hbm.at[pl.ds(off, CHUNK)], idx_v)
        pltpu.sync_copy(x_hbm.at[pl.ds(off, CHUNK)], rows_v)
        pltpu.async_copy(rows_v, out_hbm.at[idx_v], sem).wait()


def _sc_gather_body(y_hbm, pos_hbm, out_hbm, idx_v, rows_v, sem):
    wid = lax.axis_index("s") * 2 + lax.axis_index("c")
    base = wid * ROWS_PER_W
    for k in range(ROWS_PER_W // CHUNK):
        off = base + k * CHUNK
        pltpu.sync_copy(pos_hbm.at[pl.ds(off, CHUNK)], idx_v)
        pltpu.async_copy(y_hbm.at[idx_v], rows_v, sem).wait()
        pltpu.sync_copy(rows_v, out_hbm.at[pl.ds(off, CHUNK)])


@functools.lru_cache(maxsize=None)
def _sc_kernels():
    mesh = plsc.VectorSubcoreMesh(
        core_axis_name="c", subcore_axis_name="s", num_cores=2, num_subcores=16
    )
    scratch = [
        pltpu.VMEM((CHUNK,), jnp.int32),
        pltpu.VMEM((CHUNK, D), jnp.float32),
        pltpu.SemaphoreType.DMA,
    ]
    mk = functools.partial(
        pl.kernel,
        out_type=jax.ShapeDtypeStruct((B, D), jnp.float32),
        mesh=mesh,
        scratch_types=scratch,
    )
    return mk(_sc_scatter_body), mk(_sc_gather_body)


def _worklist(starts):
    i32 = jnp.int32
    ends = jnp.concatenate([starts[1:], jnp.array([B], dtype=i32)])
    counts = ends - starts
    nonempty = counts > 0
    first_blk = starts // BM
    last_blk = jnp.where(nonempty, (ends - 1) // BM, 0)
    n_items = jnp.where(nonempty, last_blk - first_blk + 1, 0)
    item_start = jnp.concatenate(
        [jnp.zeros((1,), dtype=i32), jnp.cumsum(n_items)[:-1].astype(i32)]
    )
    total = jnp.sum(n_items)
    wids = jnp.arange(MAX_WORK, dtype=i32)
    belongs = jnp.logical_and(
        wids[None, :] >= item_start[:, None],
        wids[None, :] < (item_start + n_items)[:, None],
    )
    g = jnp.argmax(belongs, axis=0).astype(i32)
    valid = wids < total
    g_last = jnp.argmax(
        jnp.where(nonempty, jnp.arange(NUM_TYPES, dtype=i32), -1)
    ).astype(i32)
    rb = jnp.where(valid, first_blk[g] + (wids - item_start[g]), NBLK - 1)
    tb = jnp.where(valid, g, g_last)
    gs = jnp.where(valid, starts[g], 0)
    ge = jnp.where(valid, ends[g], 0)
    return rb, tb, gs, ge


def kernel(x, types, W, b):
    types2d = types.reshape(SUB, LANES)
    pos2d, starts_rows = _routing(types2d)
    pos = pos2d.reshape(B)
    starts = starts_rows[:, 0]
    rb, tb, gs, ge = _worklist(starts)
    scatter_rows, gather_rows = _sc_kernels()
    x_sorted = scatter_rows(x, pos)
    y_sorted = _grouped_matmul(rb, tb, gs, ge, x_sorted, W, b)
    return gather_rows(y_sorted, pos)
